# baseline (speedup 1.0000x reference)
import functools

import jax
import jax.numpy as jnp
from jax import lax
from jax.experimental import pallas as pl
from jax.experimental.pallas import tpu as pltpu
from jax.experimental.pallas import tpu_sc as plsc

_CHUNK = 4


def _logits_kernel(x_ref, rw_ref, out_ref):
    out_ref[...] = jnp.dot(x_ref[...], rw_ref[...],
                           preferred_element_type=jnp.float32)


def _sc_route_body(logits_hbm, out_hbm, lbuf, obuf, *, rows_per_worker, n_experts):
    info = plsc.get_sparse_core_info()
    wid = lax.axis_index("s") * info.num_cores + lax.axis_index("c")
    base = wid * rows_per_worker
    pltpu.sync_copy(logits_hbm.at[pl.ds(base, rows_per_worker)], lbuf)
    iota = lax.broadcasted_iota(jnp.int32, (16,), 0)
    for t in range(rows_per_worker):
        v = lbuf[t, :]
        m1 = jnp.max(v)
        i1 = jnp.min(jnp.where(v == m1, iota, n_experts))
        masked = jnp.where(iota == i1, -jnp.inf, v)
        m2 = jnp.max(masked)
        i2 = jnp.min(jnp.where(masked == m2, iota, n_experts))
        p1 = 1.0 / (1.0 + jnp.exp(jnp.broadcast_to(m2 - m1, (16,))))
        comb = jnp.where(iota == i1, p1,
                         jnp.where(iota == i2, 1.0 - p1, 0.0))
        obuf[t, :] = comb
    pltpu.sync_copy(obuf, out_hbm.at[pl.ds(base, rows_per_worker)])


def _moe_kernel(x_ref, wcomb_ref, bcomb_ref, wbank_hbm, bbank_ref, out_ref,
                wbuf, sems, *, n_experts):
    n_chunks = n_experts // _CHUNK
    for c in range(n_chunks):
        sl = pl.ds(c * _CHUNK, _CHUNK)
        pltpu.make_async_copy(wbank_hbm.at[sl], wbuf.at[sl], sems.at[c]).start()

    x = x_ref[...]
    w_comb = wcomb_ref[...]
    acc = jnp.dot(bcomb_ref[...], bbank_ref[...],
                  preferred_element_type=jnp.float32)
    for c in range(n_chunks):
        sl = pl.ds(c * _CHUNK, _CHUNK)
        pltpu.make_async_copy(wbank_hbm.at[sl], wbuf.at[sl], sems.at[c]).wait()
        for e in range(c * _CHUNK, (c + 1) * _CHUNK):
            y = jnp.dot(x, wbuf[e], preferred_element_type=jnp.float32)
            acc = acc + w_comb[:, e][:, None] * y
    out_ref[...] = acc


@jax.jit
def kernel(input_batch, router_w, bias_router_w, weight_bank, bias_bank):
    b, d = input_batch.shape
    e, _, o = weight_bank.shape

    rw_cat = jnp.concatenate([router_w, bias_router_w], axis=1)
    logits = pl.pallas_call(
        _logits_kernel,
        out_shape=jax.ShapeDtypeStruct((b, 2 * e), jnp.float32),
        in_specs=[
            pl.BlockSpec((b, d), lambda: (0, 0)),
            pl.BlockSpec((d, 2 * e), lambda: (0, 0)),
        ],
        out_specs=pl.BlockSpec((b, 2 * e), lambda: (0, 0)),
    )(input_batch, rw_cat)

    rows = jnp.concatenate([logits[:, :e], logits[:, e:]], axis=0)

    info = plsc.get_sparse_core_info()
    n_workers = info.num_cores * info.num_subcores
    rpw = (2 * b) // n_workers
    mesh = plsc.VectorSubcoreMesh(core_axis_name="c", subcore_axis_name="s")
    comb = pl.kernel(
        functools.partial(_sc_route_body, rows_per_worker=rpw, n_experts=e),
        out_type=jax.ShapeDtypeStruct((2 * b, e), jnp.float32),
        mesh=mesh,
        scratch_types=[
            pltpu.VMEM((rpw, e), jnp.float32),
            pltpu.VMEM((rpw, e), jnp.float32),
        ],
        compiler_params=pltpu.CompilerParams(needs_layout_passes=False),
    )(rows)

    w_comb, b_comb = comb[:b], comb[b:]

    return pl.pallas_call(
        functools.partial(_moe_kernel, n_experts=e),
        out_shape=jax.ShapeDtypeStruct((b, o), jnp.float32),
        in_specs=[
            pl.BlockSpec((b, d), lambda: (0, 0)),
            pl.BlockSpec((b, e), lambda: (0, 0)),
            pl.BlockSpec((b, e), lambda: (0, 0)),
            pl.BlockSpec(memory_space=pl.ANY),
            pl.BlockSpec((e, o), lambda: (0, 0)),
        ],
        out_specs=pl.BlockSpec((b, o), lambda: (0, 0)),
        scratch_shapes=[
            pltpu.VMEM((e, d, o), jnp.float32),
            pltpu.SemaphoreType.DMA((e // _CHUNK,)),
        ],
    )(input_batch, w_comb, b_comb, weight_bank, bias_bank)


# speedup vs baseline: 2.9005x; 2.9005x over previous
import functools

import jax
import jax.numpy as jnp
from jax.experimental import pallas as pl
from jax.experimental.pallas import tpu as pltpu

_CHUNKS = (5, 5, 4, 2)


def _top2_combine(logits, e):
    iota = jax.lax.broadcasted_iota(jnp.int32, logits.shape, 1)
    m1 = jnp.max(logits, axis=-1, keepdims=True)
    i1 = jnp.min(jnp.where(logits == m1, iota, e), axis=-1, keepdims=True)
    masked = jnp.where(iota == i1, -jnp.inf, logits)
    m2 = jnp.max(masked, axis=-1, keepdims=True)
    i2 = jnp.min(jnp.where(masked == m2, iota, e), axis=-1, keepdims=True)
    p1 = jax.nn.sigmoid(m1 - m2)
    p2 = 1.0 - p1
    return jnp.where(iota == i1, p1, 0.0) + jnp.where(iota == i2, p2, 0.0)


def _moe_kernel(x_ref, rw_ref, brw_ref, wbank_hbm, bbank_ref, out_ref,
                wbuf, sems, *, n_experts):
    starts = [sum(_CHUNKS[:c]) for c in range(len(_CHUNKS))]
    for c, (s0, w) in enumerate(zip(starts, _CHUNKS)):
        sl = pl.ds(s0, w)
        pltpu.make_async_copy(wbank_hbm.at[sl], wbuf.at[sl], sems.at[c]).start()

    x = x_ref[...]
    w_logits = jnp.dot(x, rw_ref[...], preferred_element_type=jnp.float32)
    b_logits = jnp.dot(x, brw_ref[...], preferred_element_type=jnp.float32)
    w_comb = _top2_combine(w_logits, n_experts)
    b_comb = _top2_combine(b_logits, n_experts)

    acc = jnp.dot(b_comb, bbank_ref[...], preferred_element_type=jnp.float32)
    for c, (s0, w) in enumerate(zip(starts, _CHUNKS)):
        sl = pl.ds(s0, w)
        pltpu.make_async_copy(wbank_hbm.at[sl], wbuf.at[sl], sems.at[c]).wait()
        for e in range(s0, s0 + w):
            y = jnp.dot(x, wbuf[e], preferred_element_type=jnp.float32)
            acc = acc + w_comb[:, e][:, None] * y
    out_ref[...] = acc


@jax.jit
def kernel(input_batch, router_w, bias_router_w, weight_bank, bias_bank):
    b, d = input_batch.shape
    e, _, o = weight_bank.shape
    return pl.pallas_call(
        functools.partial(_moe_kernel, n_experts=e),
        out_shape=jax.ShapeDtypeStruct((b, o), jnp.float32),
        in_specs=[
            pl.BlockSpec((b, d), lambda: (0, 0)),
            pl.BlockSpec((d, e), lambda: (0, 0)),
            pl.BlockSpec((d, e), lambda: (0, 0)),
            pl.BlockSpec(memory_space=pl.ANY),
            pl.BlockSpec((e, o), lambda: (0, 0)),
        ],
        out_specs=pl.BlockSpec((b, o), lambda: (0, 0)),
        scratch_shapes=[
            pltpu.VMEM((e, d, o), jnp.float32),
            pltpu.SemaphoreType.DMA((len(_CHUNKS),)),
        ],
    )(input_batch, router_w, bias_router_w, weight_bank, bias_bank)


# speedup vs baseline: 2.9418x; 1.0142x over previous
import functools

import jax
import jax.numpy as jnp
from jax.experimental import pallas as pl
from jax.experimental.pallas import tpu as pltpu

_CHUNKS = (4, 4, 4, 2, 2)


def _top2_combine(logits, e):
    iota = jax.lax.broadcasted_iota(jnp.int32, logits.shape, 1)
    m1 = jnp.max(logits, axis=-1, keepdims=True)
    i1 = jnp.min(jnp.where(logits == m1, iota, e), axis=-1, keepdims=True)
    masked = jnp.where(iota == i1, -jnp.inf, logits)
    m2 = jnp.max(masked, axis=-1, keepdims=True)
    i2 = jnp.min(jnp.where(masked == m2, iota, e), axis=-1, keepdims=True)
    p1 = jax.nn.sigmoid(m1 - m2)
    p2 = 1.0 - p1
    return jnp.where(iota == i1, p1, 0.0) + jnp.where(iota == i2, p2, 0.0)


def _moe_kernel(x_ref, rw_ref, brw_ref, wbank_hbm, bbank_ref, out_ref,
                wbuf, sems, *, n_experts):
    starts = [sum(_CHUNKS[:c]) for c in range(len(_CHUNKS))]
    for c, (s0, w) in enumerate(zip(starts, _CHUNKS)):
        sl = pl.ds(s0, w)
        pltpu.make_async_copy(wbank_hbm.at[sl], wbuf.at[sl], sems.at[c]).start()

    x = x_ref[...]
    w_logits = jnp.dot(x, rw_ref[...], preferred_element_type=jnp.float32)
    b_logits = jnp.dot(x, brw_ref[...], preferred_element_type=jnp.float32)
    w_comb = _top2_combine(w_logits, n_experts)
    b_comb = _top2_combine(b_logits, n_experts)

    acc = jnp.dot(b_comb, bbank_ref[...], preferred_element_type=jnp.float32)
    for c, (s0, w) in enumerate(zip(starts, _CHUNKS)):
        sl = pl.ds(s0, w)
        pltpu.make_async_copy(wbank_hbm.at[sl], wbuf.at[sl], sems.at[c]).wait()
        for e in range(s0, s0 + w):
            y = jnp.dot(x, wbuf[e], preferred_element_type=jnp.float32)
            acc = acc + w_comb[:, e][:, None] * y
    out_ref[...] = acc


@jax.jit
def kernel(input_batch, router_w, bias_router_w, weight_bank, bias_bank):
    b, d = input_batch.shape
    e, _, o = weight_bank.shape
    return pl.pallas_call(
        functools.partial(_moe_kernel, n_experts=e),
        out_shape=jax.ShapeDtypeStruct((b, o), jnp.float32),
        in_specs=[
            pl.BlockSpec((b, d), lambda: (0, 0)),
            pl.BlockSpec((d, e), lambda: (0, 0)),
            pl.BlockSpec((d, e), lambda: (0, 0)),
            pl.BlockSpec(memory_space=pl.ANY),
            pl.BlockSpec((e, o), lambda: (0, 0)),
        ],
        out_specs=pl.BlockSpec((b, o), lambda: (0, 0)),
        scratch_shapes=[
            pltpu.VMEM((e, d, o), jnp.float32),
            pltpu.SemaphoreType.DMA((len(_CHUNKS),)),
        ],
    )(input_batch, router_w, bias_router_w, weight_bank, bias_bank)


# speedup vs baseline: 2.9926x; 1.0172x over previous
import functools

import jax
import jax.numpy as jnp
from jax.experimental import pallas as pl
from jax.experimental.pallas import tpu as pltpu

_CHUNK = 4


def _top2_combine(logits, e):
    iota = jax.lax.broadcasted_iota(jnp.int32, logits.shape, 1)
    m1 = jnp.max(logits, axis=-1, keepdims=True)
    i1 = jnp.min(jnp.where(logits == m1, iota, e), axis=-1, keepdims=True)
    masked = jnp.where(iota == i1, -jnp.inf, logits)
    m2 = jnp.max(masked, axis=-1, keepdims=True)
    i2 = jnp.min(jnp.where(masked == m2, iota, e), axis=-1, keepdims=True)
    p1 = jax.nn.sigmoid(m1 - m2)
    p2 = 1.0 - p1
    return jnp.where(iota == i1, p1, 0.0) + jnp.where(iota == i2, p2, 0.0)


def _moe_kernel(x_ref, rw_ref, brw_ref, wbank_hbm, bbank_ref, out_ref,
                wbuf, sems, *, n_experts):
    n_chunks = n_experts // _CHUNK
    for c in range(n_chunks):
        sl = pl.ds(c * _CHUNK, _CHUNK)
        pltpu.make_async_copy(wbank_hbm.at[sl], wbuf.at[sl], sems.at[c]).start()

    x = x_ref[...]
    w_logits = jnp.dot(x, rw_ref[...], preferred_element_type=jnp.float32)
    b_logits = jnp.dot(x, brw_ref[...], preferred_element_type=jnp.float32)
    w_comb = _top2_combine(w_logits, n_experts)
    b_comb = _top2_combine(b_logits, n_experts)

    acc = jnp.dot(b_comb, bbank_ref[...], preferred_element_type=jnp.float32)
    for c in range(n_chunks):
        sl = pl.ds(c * _CHUNK, _CHUNK)
        pltpu.make_async_copy(wbank_hbm.at[sl], wbuf.at[sl], sems.at[c]).wait()
        for e in range(c * _CHUNK, (c + 1) * _CHUNK):
            y = jnp.dot(x, wbuf[e], preferred_element_type=jnp.float32)
            acc = acc + w_comb[:, e][:, None] * y
    out_ref[...] = acc


@jax.jit
def kernel(input_batch, router_w, bias_router_w, weight_bank, bias_bank):
    b, d = input_batch.shape
    e, _, o = weight_bank.shape
    return pl.pallas_call(
        functools.partial(_moe_kernel, n_experts=e),
        out_shape=jax.ShapeDtypeStruct((b, o), jnp.float32),
        in_specs=[
            pl.BlockSpec((b, d), lambda: (0, 0)),
            pl.BlockSpec((d, e), lambda: (0, 0)),
            pl.BlockSpec((d, e), lambda: (0, 0)),
            pl.BlockSpec(memory_space=pl.ANY),
            pl.BlockSpec((e, o), lambda: (0, 0)),
        ],
        out_specs=pl.BlockSpec((b, o), lambda: (0, 0)),
        scratch_shapes=[
            pltpu.VMEM((e, d, o), jnp.float32),
            pltpu.SemaphoreType.DMA((e // _CHUNK,)),
        ],
    )(input_batch, router_w, bias_router_w, weight_bank, bias_bank)
